# Initial kernel scaffold; baseline (speedup 1.0000x reference)
import jax
import jax.numpy as jnp
from jax.experimental import pallas as pl


def kernel(input, dates, cmax, time_w, time_b, local_emb, space_emb):
    raise NotImplementedError("write your pallas kernel here")


# speedup vs baseline: 1.5241x; 1.5241x over previous
import jax
import jax.numpy as jnp
from jax.experimental import pallas as pl
from jax.experimental.pallas import tpu as pltpu

B, L, D_IN = 8, 2048, 16
N_TIME, PER_DIM = 6, 6
D_MODEL = 40


def _body(inp_ref, dates_ref, cmax_ref, tw_ref, tb_ref, le_ref, se_ref,
          out_ref, vid_ref, base_ref):
    d = pl.program_id(1)

    @pl.when(d == 0)
    def _compute_base():
        dates = dates_ref[0]
        pieces = []
        for i in range(N_TIME):
            xa = dates[:, i:i + 1] * tw_ref[i:i + 1, :] + tb_ref[i:i + 1, :]
            pieces.append(xa)
        xa = jnp.concatenate(pieces, axis=1)
        k = jax.lax.broadcasted_iota(jnp.int32, (L, N_TIME * PER_DIM), 1)
        time_emb = jnp.where(k % PER_DIM == 0, xa, jnp.sin(xa))
        base = jnp.concatenate(
            [le_ref[:, 0:1],
             time_emb + le_ref[:, 1:1 + N_TIME * PER_DIM],
             cmax_ref[0] + le_ref[:, 1 + N_TIME * PER_DIM:]], axis=1)
        base_ref[...] = base

    inp = inp_ref[0]
    lane = jax.lax.broadcasted_iota(jnp.int32, (L, D_IN), 1)
    val = jnp.sum(jnp.where(lane == d, inp, 0.0), axis=1, keepdims=True)
    valsp = val + se_ref[d]
    c = jax.lax.broadcasted_iota(jnp.int32, (L, D_MODEL), 1)
    out_ref[0] = base_ref[...] + jnp.where(
        c == 0, jnp.broadcast_to(valsp, (L, D_MODEL)), 0.0)
    vid_ref[...] = jnp.full((1, 1, 1, L), d, dtype=jnp.int32)


def kernel(input, dates, cmax, time_w, time_b, local_emb, space_emb):
    b, length, d_input = input.shape
    d_model = local_emb.shape[1]
    out, vid = pl.pallas_call(
        _body,
        grid=(b, d_input),
        in_specs=[
            pl.BlockSpec((1, length, d_input), lambda bb, dd: (bb, 0, 0)),
            pl.BlockSpec((1, length, N_TIME), lambda bb, dd: (bb, 0, 0)),
            pl.BlockSpec((1, length, 3), lambda bb, dd: (bb, 0, 0)),
            pl.BlockSpec((N_TIME, PER_DIM), lambda bb, dd: (0, 0)),
            pl.BlockSpec((N_TIME, PER_DIM), lambda bb, dd: (0, 0)),
            pl.BlockSpec((length, d_model), lambda bb, dd: (0, 0)),
            pl.BlockSpec(memory_space=pltpu.SMEM),
        ],
        out_specs=[
            pl.BlockSpec((1, length, d_model), lambda bb, dd: (bb, dd, 0)),
            pl.BlockSpec((1, 1, 1, length), lambda bb, dd: (bb, dd, 0, 0)),
        ],
        out_shape=[
            jax.ShapeDtypeStruct((b, d_input * length, d_model), jnp.float32),
            jax.ShapeDtypeStruct((b, d_input, 1, length), jnp.int32),
        ],
        scratch_shapes=[pltpu.VMEM((length, d_model), jnp.float32)],
        compiler_params=pltpu.CompilerParams(
            dimension_semantics=("arbitrary", "arbitrary")),
    )(input, dates, cmax, time_w, time_b, local_emb,
      space_emb.reshape(d_input))
    return out, vid.reshape(b, d_input * length)
